# baseline (speedup 1.0000x reference)
import dataclasses
import functools

import jax
import jax.numpy as jnp
from jax import lax
from jax.experimental import pallas as pl
from jax.experimental.pallas import tpu as pltpu
from jax.experimental.pallas import tpu_sc as plsc

N_NODES = 10000
N_EDGES = 320000
D_IN = 128
D_H = 128
D_OUT = 64

N_PAD = 10240
NUM_WORKERS = 32
CHUNK = 128
CHUNKS_W = 80
E_PAD = NUM_WORKERS * CHUNKS_W * CHUNK
STRIPE = N_PAD // 16

CH0, CH1 = 160, 0
PASSES0 = (64, 64, 32)
PASSES1 = ()
P_MAX = 64
CORE1_BASE = 16 * CH0

def _sc_params():
    cp = pltpu.CompilerParams()
    if "needs_layout_passes" in pltpu.CompilerParams.__dataclass_fields__:
        cp = dataclasses.replace(cp, needs_layout_passes=False)
    return cp


@functools.cache
def _sc_mesh():
    return plsc.VectorSubcoreMesh(
        core_axis_name="c", subcore_axis_name="s", num_cores=2, num_subcores=16
    )


def _sc_degrees(srcb, dstb):
    @functools.partial(
        pl.kernel,
        out_type=jax.ShapeDtypeStruct((NUM_WORKERS, 2, N_PAD), jnp.float32),
        mesh=_sc_mesh(),
        compiler_params=_sc_params(),
        scratch_types=[
            pltpu.VMEM((CHUNKS_W, CHUNK), jnp.int32),
            pltpu.VMEM((CHUNKS_W, CHUNK), jnp.int32),
            pltpu.VMEM((N_PAD,), jnp.float32),
            pltpu.VMEM((N_PAD,), jnp.float32),
        ],
    )
    def deg_kernel(srcb_hbm, dstb_hbm, out_hbm, src_v, dst_v, degs_v, degd_v):
        cid = lax.axis_index("c")
        sid = lax.axis_index("s")
        wid = sid * 2 + cid
        zeros16 = jnp.zeros((16,), jnp.float32)
        ones16 = jnp.ones((16,), jnp.float32)

        @pl.loop(0, N_PAD // 16)
        def _(i):
            degs_v[pl.ds(i * 16, 16)] = zeros16
            degd_v[pl.ds(i * 16, 16)] = zeros16

        pltpu.sync_copy(srcb_hbm.at[wid], src_v)
        pltpu.sync_copy(dstb_hbm.at[wid], dst_v)

        @pl.loop(0, CHUNKS_W)
        def _(j):
            for s2 in range(CHUNK // 16):
                s16 = src_v[j, pl.ds(s2 * 16, 16)]
                plsc.addupdate_scatter(degs_v, [s16], ones16)
                d16 = dst_v[j, pl.ds(s2 * 16, 16)]
                plsc.addupdate_scatter(degd_v, [d16], ones16)

        pltpu.sync_copy(degs_v, out_hbm.at[wid, 0])
        pltpu.sync_copy(degd_v, out_hbm.at[wid, 1])

    return deg_kernel(srcb, dstb)


def _sc_edge_agg(h, srcb, dstb, d):
    @functools.partial(
        pl.kernel,
        out_type=jax.ShapeDtypeStruct((N_PAD, d), jnp.float32),
        mesh=_sc_mesh(),
        scratch_types=[
            pltpu.VMEM((P_MAX, CHUNK), jnp.int32),
            pltpu.VMEM((P_MAX, CHUNK), jnp.int32),
            pltpu.VMEM((CHUNK, d), jnp.float32),
            pltpu.VMEM_SHARED((N_PAD, d), jnp.float32),
        ],
    )
    def agg_kernel(h_hbm, srcb_hbm, dstb_hbm, out_hbm, src_v, dst_v, msgs_v, agg_sh):
        cid = lax.axis_index("c")
        sid = lax.axis_index("s")
        zeros16 = jnp.zeros((16,), jnp.float32)
        base = sid * STRIPE

        @pl.when(cid == 0)
        def _():
            @pl.loop(0, CHUNK)
            def _(r):
                for s2 in range(d // 16):
                    msgs_v[r, pl.ds(s2 * 16, 16)] = zeros16

            for z in range(STRIPE // CHUNK):
                pltpu.sync_copy(msgs_v, agg_sh.at[pl.ds(base + z * CHUNK, CHUNK)])
        plsc.subcore_barrier()

        def edge_pass(chunk0, n_chunks):
            pltpu.sync_copy(srcb_hbm.at[pl.ds(chunk0, n_chunks)],
                            src_v.at[pl.ds(0, n_chunks)])
            pltpu.sync_copy(dstb_hbm.at[pl.ds(chunk0, n_chunks)],
                            dst_v.at[pl.ds(0, n_chunks)])

            @pl.loop(0, n_chunks)
            def _(j):
                pltpu.sync_copy(h_hbm.at[src_v.at[j]], msgs_v)
                pltpu.sync_copy(msgs_v, agg_sh.at[dst_v.at[j]], add=True)

        @pl.when(cid == 0)
        def _():
            off = 0
            for n in PASSES0:
                edge_pass(sid * CH0 + off, n)
                off += n

        if PASSES1:
            @pl.when(cid == 1)
            def _():
                off = 0
                for n in PASSES1:
                    edge_pass(CORE1_BASE + sid * CH1 + off, n)
                    off += n

        plsc.subcore_barrier()

        @pl.when(cid == 0)
        def _():
            pltpu.sync_copy(
                agg_sh.at[pl.ds(base, STRIPE)], out_hbm.at[pl.ds(base, STRIPE)]
            )

    return agg_kernel(h, srcb, dstb)


def _norm_from_deg(deg):
    return jnp.where(deg > 0, lax.rsqrt(jnp.maximum(deg, 1e-12)), 0.0)


_DOT = functools.partial(
    lax.dot_general,
    dimension_numbers=(((1,), (0,)), ((), ())),
    precision=lax.Precision.HIGHEST,
    preferred_element_type=jnp.float32,
)

_BLK = 1280


def _tc_prep(xp, degp, W1):
    def body(x_ref, degp_ref, w_ref, out_ref):
        deg = jnp.sum(degp_ref[...], axis=0)
        norm_out = _norm_from_deg(deg[0])
        xs = x_ref[...] * norm_out[:, None]
        out_ref[...] = _DOT(xs, w_ref[...])

    return pl.pallas_call(
        body,
        grid=(N_PAD // _BLK,),
        in_specs=[
            pl.BlockSpec((_BLK, D_IN), lambda i: (i, 0)),
            pl.BlockSpec((NUM_WORKERS, 2, _BLK), lambda i: (0, 0, i)),
            pl.BlockSpec((D_IN, D_H), lambda i: (0, 0)),
        ],
        out_specs=pl.BlockSpec((_BLK, D_H), lambda i: (i, 0)),
        out_shape=jax.ShapeDtypeStruct((N_PAD, D_H), jnp.float32),
    )(xp, degp, W1)


def _tc_mid(aggp, degp, b1, W2):
    def body(aggp_ref, degp_ref, b1_ref, w_ref, out_ref):
        agg = aggp_ref[...]
        deg = jnp.sum(degp_ref[...], axis=0)
        norm_out = _norm_from_deg(deg[0])
        norm_in = _norm_from_deg(deg[1])
        h = jnp.maximum(agg * norm_in[:, None] + b1_ref[...], 0.0)
        h = h * norm_out[:, None]
        out_ref[...] = _DOT(h, w_ref[...])

    return pl.pallas_call(
        body,
        grid=(N_PAD // _BLK,),
        in_specs=[
            pl.BlockSpec((_BLK, D_H), lambda i: (i, 0)),
            pl.BlockSpec((NUM_WORKERS, 2, _BLK), lambda i: (0, 0, i)),
            pl.BlockSpec((1, D_H), lambda i: (0, 0)),
            pl.BlockSpec((D_H, D_H), lambda i: (0, 0)),
        ],
        out_specs=pl.BlockSpec((_BLK, D_H), lambda i: (i, 0)),
        out_shape=jax.ShapeDtypeStruct((N_PAD, D_H), jnp.float32),
    )(aggp, degp, b1, W2)


def _tc_final(aggp, degp, b2):
    def body(aggp_ref, degp_ref, b2_ref, out_ref):
        agg = aggp_ref[:, :D_OUT]
        deg = jnp.sum(degp_ref[...], axis=0)
        norm_in = _norm_from_deg(deg[1])
        out_ref[...] = agg * norm_in[:, None] + b2_ref[...]

    return pl.pallas_call(
        body,
        grid=(N_PAD // _BLK,),
        in_specs=[
            pl.BlockSpec((_BLK, D_H), lambda i: (i, 0)),
            pl.BlockSpec((NUM_WORKERS, 2, _BLK), lambda i: (0, 0, i)),
            pl.BlockSpec((1, D_OUT), lambda i: (0, 0)),
        ],
        out_specs=pl.BlockSpec((_BLK, D_OUT), lambda i: (i, 0)),
        out_shape=jax.ShapeDtypeStruct((N_PAD, D_OUT), jnp.float32),
    )(aggp, degp, b2)


def kernel(features, edge_index, W1, b1, W2, b2):
    xp = jnp.zeros((N_PAD, D_IN), jnp.float32).at[:N_NODES].set(features)
    pad = jnp.full((2, E_PAD - N_EDGES), N_NODES, jnp.int32)
    ei = jnp.concatenate([edge_index, pad], axis=1)
    srcb, dstb = ei[0].reshape(NUM_WORKERS, CHUNKS_W, CHUNK), ei[1].reshape(
        NUM_WORKERS, CHUNKS_W, CHUNK)
    srcf, dstf = ei[0].reshape(-1, CHUNK), ei[1].reshape(-1, CHUNK)

    W2p = jnp.concatenate([W2, jnp.zeros((D_H, D_H - D_OUT), jnp.float32)], axis=1)

    degp = _sc_degrees(srcb, dstb)
    h1p = _tc_prep(xp, degp, W1)
    agg1 = _sc_edge_agg(h1p, srcf, dstf, D_H)
    h2p = _tc_mid(agg1, degp, b1.reshape(1, D_H), W2p)
    agg2 = _sc_edge_agg(h2p, srcf, dstf, D_H)
    outp = _tc_final(agg2, degp, b2.reshape(1, D_OUT))
    return outp[:N_NODES]


# speedup vs baseline: 3.1020x; 3.1020x over previous
import dataclasses
import functools

import jax
import jax.numpy as jnp
from jax import lax
from jax.experimental import pallas as pl
from jax.experimental.pallas import tpu as pltpu
from jax.experimental.pallas import tpu_sc as plsc

N_NODES = 10000
N_EDGES = 320000
D_IN = 128
D_H = 128
D_OUT = 64

N_PAD = 10240
NUM_WORKERS = 32
CHUNK = 128
CHUNKS_W = 80
E_PAD = NUM_WORKERS * CHUNKS_W * CHUNK
STRIPE = N_PAD // 16

CH_W = 80
CORE1_BASE = 16 * CH_W

def _sc_params():
    cp = pltpu.CompilerParams()
    if "needs_layout_passes" in pltpu.CompilerParams.__dataclass_fields__:
        cp = dataclasses.replace(cp, needs_layout_passes=False)
    return cp


@functools.cache
def _sc_mesh():
    return plsc.VectorSubcoreMesh(
        core_axis_name="c", subcore_axis_name="s", num_cores=2, num_subcores=16
    )


def _sc_degrees(srcb, dstb):
    @functools.partial(
        pl.kernel,
        out_type=jax.ShapeDtypeStruct((NUM_WORKERS, 2, N_PAD), jnp.float32),
        mesh=_sc_mesh(),
        compiler_params=_sc_params(),
        scratch_types=[
            pltpu.VMEM((CHUNKS_W, CHUNK), jnp.int32),
            pltpu.VMEM((CHUNKS_W, CHUNK), jnp.int32),
            pltpu.VMEM((N_PAD,), jnp.float32),
            pltpu.VMEM((N_PAD,), jnp.float32),
        ],
    )
    def deg_kernel(srcb_hbm, dstb_hbm, out_hbm, src_v, dst_v, degs_v, degd_v):
        cid = lax.axis_index("c")
        sid = lax.axis_index("s")
        wid = sid * 2 + cid
        zeros16 = jnp.zeros((16,), jnp.float32)
        ones16 = jnp.ones((16,), jnp.float32)

        @pl.loop(0, N_PAD // 16)
        def _(i):
            degs_v[pl.ds(i * 16, 16)] = zeros16
            degd_v[pl.ds(i * 16, 16)] = zeros16

        pltpu.sync_copy(srcb_hbm.at[wid], src_v)
        pltpu.sync_copy(dstb_hbm.at[wid], dst_v)

        @pl.loop(0, CHUNKS_W)
        def _(j):
            for s2 in range(CHUNK // 16):
                s16 = src_v[j, pl.ds(s2 * 16, 16)]
                plsc.addupdate_scatter(degs_v, [s16], ones16)
                d16 = dst_v[j, pl.ds(s2 * 16, 16)]
                plsc.addupdate_scatter(degd_v, [d16], ones16)

        pltpu.sync_copy(degs_v, out_hbm.at[wid, 0])
        pltpu.sync_copy(degd_v, out_hbm.at[wid, 1])

    return deg_kernel(srcb, dstb)


def _sc_edge_agg(h, srcb, dstb, d):
    @functools.partial(
        pl.kernel,
        out_type=jax.ShapeDtypeStruct((2, N_PAD, d), jnp.float32),
        mesh=_sc_mesh(),
        scratch_types=[
            pltpu.VMEM((CH_W, CHUNK), jnp.int32),
            pltpu.VMEM((CH_W, CHUNK), jnp.int32),
            pltpu.VMEM((CHUNK, d), jnp.float32),
            pltpu.VMEM_SHARED((N_PAD, d), jnp.float32),
        ],
    )
    def agg_kernel(h_hbm, srcb_hbm, dstb_hbm, out_hbm, src_v, dst_v, msgs_v, agg_sh):
        cid = lax.axis_index("c")
        sid = lax.axis_index("s")
        zeros16 = jnp.zeros((16,), jnp.float32)
        base = sid * STRIPE

        @pl.loop(0, CHUNK)
        def _(r):
            for s2 in range(d // 16):
                msgs_v[r, pl.ds(s2 * 16, 16)] = zeros16

        for z in range(STRIPE // CHUNK):
            pltpu.sync_copy(msgs_v, agg_sh.at[pl.ds(base + z * CHUNK, CHUNK)])
        plsc.subcore_barrier()

        chunk0 = cid * CORE1_BASE + sid * CH_W
        pltpu.sync_copy(srcb_hbm.at[pl.ds(chunk0, CH_W)], src_v)
        pltpu.sync_copy(dstb_hbm.at[pl.ds(chunk0, CH_W)], dst_v)

        @pl.loop(0, CH_W)
        def _(j):
            pltpu.sync_copy(h_hbm.at[src_v.at[j]], msgs_v)
            pltpu.sync_copy(msgs_v, agg_sh.at[dst_v.at[j]], add=True)

        plsc.subcore_barrier()
        pltpu.sync_copy(
            agg_sh.at[pl.ds(base, STRIPE)], out_hbm.at[cid, pl.ds(base, STRIPE)]
        )

    return agg_kernel(h, srcb, dstb)


def _norm_from_deg(deg):
    return jnp.where(deg > 0, lax.rsqrt(jnp.maximum(deg, 1e-12)), 0.0)


_DOT = functools.partial(
    lax.dot_general,
    dimension_numbers=(((1,), (0,)), ((), ())),
    precision=lax.Precision.HIGHEST,
    preferred_element_type=jnp.float32,
)

_BLK = 1280


def _tc_prep(xp, degp, W1):
    def body(x_ref, degp_ref, w_ref, out_ref):
        deg = jnp.sum(degp_ref[...], axis=0)
        norm_out = _norm_from_deg(deg[0])
        xs = x_ref[...] * norm_out[:, None]
        out_ref[...] = _DOT(xs, w_ref[...])

    return pl.pallas_call(
        body,
        grid=(N_PAD // _BLK,),
        in_specs=[
            pl.BlockSpec((_BLK, D_IN), lambda i: (i, 0)),
            pl.BlockSpec((NUM_WORKERS, 2, _BLK), lambda i: (0, 0, i)),
            pl.BlockSpec((D_IN, D_H), lambda i: (0, 0)),
        ],
        out_specs=pl.BlockSpec((_BLK, D_H), lambda i: (i, 0)),
        out_shape=jax.ShapeDtypeStruct((N_PAD, D_H), jnp.float32),
    )(xp, degp, W1)


def _tc_mid(aggp, degp, b1, W2):
    def body(aggp_ref, degp_ref, b1_ref, w_ref, out_ref):
        agg = aggp_ref[0] + aggp_ref[1]
        deg = jnp.sum(degp_ref[...], axis=0)
        norm_out = _norm_from_deg(deg[0])
        norm_in = _norm_from_deg(deg[1])
        h = jnp.maximum(agg * norm_in[:, None] + b1_ref[...], 0.0)
        h = h * norm_out[:, None]
        out_ref[...] = _DOT(h, w_ref[...])

    return pl.pallas_call(
        body,
        grid=(N_PAD // _BLK,),
        in_specs=[
            pl.BlockSpec((2, _BLK, D_H), lambda i: (0, i, 0)),
            pl.BlockSpec((NUM_WORKERS, 2, _BLK), lambda i: (0, 0, i)),
            pl.BlockSpec((1, D_H), lambda i: (0, 0)),
            pl.BlockSpec((D_H, D_H), lambda i: (0, 0)),
        ],
        out_specs=pl.BlockSpec((_BLK, D_H), lambda i: (i, 0)),
        out_shape=jax.ShapeDtypeStruct((N_PAD, D_H), jnp.float32),
    )(aggp, degp, b1, W2)


def _tc_final(aggp, degp, b2):
    def body(aggp_ref, degp_ref, b2_ref, out_ref):
        agg = aggp_ref[0, :, :D_OUT] + aggp_ref[1, :, :D_OUT]
        deg = jnp.sum(degp_ref[...], axis=0)
        norm_in = _norm_from_deg(deg[1])
        out_ref[...] = agg * norm_in[:, None] + b2_ref[...]

    return pl.pallas_call(
        body,
        grid=(N_PAD // _BLK,),
        in_specs=[
            pl.BlockSpec((2, _BLK, D_H), lambda i: (0, i, 0)),
            pl.BlockSpec((NUM_WORKERS, 2, _BLK), lambda i: (0, 0, i)),
            pl.BlockSpec((1, D_OUT), lambda i: (0, 0)),
        ],
        out_specs=pl.BlockSpec((_BLK, D_OUT), lambda i: (i, 0)),
        out_shape=jax.ShapeDtypeStruct((N_PAD, D_OUT), jnp.float32),
    )(aggp, degp, b2)


def kernel(features, edge_index, W1, b1, W2, b2):
    xp = jnp.zeros((N_PAD, D_IN), jnp.float32).at[:N_NODES].set(features)
    pad_ids = N_NODES + jnp.arange(E_PAD - N_EDGES, dtype=jnp.int32) % (
        N_PAD - N_NODES)
    ei = jnp.concatenate([edge_index, jnp.stack([pad_ids, pad_ids])], axis=1)
    srcb, dstb = ei[0].reshape(NUM_WORKERS, CHUNKS_W, CHUNK), ei[1].reshape(
        NUM_WORKERS, CHUNKS_W, CHUNK)
    srcf, dstf = ei[0].reshape(-1, CHUNK), ei[1].reshape(-1, CHUNK)

    W2p = jnp.concatenate([W2, jnp.zeros((D_H, D_H - D_OUT), jnp.float32)], axis=1)

    degp = _sc_degrees(srcb, dstb)
    h1p = _tc_prep(xp, degp, W1)
    agg1 = _sc_edge_agg(h1p, srcf, dstf, D_H)
    h2p = _tc_mid(agg1, degp, b1.reshape(1, D_H), W2p)
    agg2 = _sc_edge_agg(h2p, srcf, dstf, D_H)
    outp = _tc_final(agg2, degp, b2.reshape(1, D_OUT))
    return outp[:N_NODES]


# speedup vs baseline: 3.8063x; 1.2271x over previous
import dataclasses
import functools

import jax
import jax.numpy as jnp
from jax import lax
from jax.experimental import pallas as pl
from jax.experimental.pallas import tpu as pltpu
from jax.experimental.pallas import tpu_sc as plsc

N_NODES = 10000
N_EDGES = 320000
D_IN = 128
D_H = 128
D_OUT = 64

N_PAD = 10240
NUM_WORKERS = 32
CHUNK = 128
CHUNKS_W = 80
E_PAD = NUM_WORKERS * CHUNKS_W * CHUNK
STRIPE = N_PAD // 16

CH_W = 80
PASS_N = 16
CORE1_BASE = 16 * CH_W

def _sc_params():
    cp = pltpu.CompilerParams()
    if "needs_layout_passes" in pltpu.CompilerParams.__dataclass_fields__:
        cp = dataclasses.replace(cp, needs_layout_passes=False)
    return cp


@functools.cache
def _sc_mesh():
    return plsc.VectorSubcoreMesh(
        core_axis_name="c", subcore_axis_name="s", num_cores=2, num_subcores=16
    )


def _sc_degrees(srcb, dstb):
    @functools.partial(
        pl.kernel,
        out_type=jax.ShapeDtypeStruct((NUM_WORKERS, 2, N_PAD), jnp.float32),
        mesh=_sc_mesh(),
        compiler_params=_sc_params(),
        scratch_types=[
            pltpu.VMEM((CHUNKS_W, CHUNK), jnp.int32),
            pltpu.VMEM((CHUNKS_W, CHUNK), jnp.int32),
            pltpu.VMEM((N_PAD,), jnp.float32),
            pltpu.VMEM((N_PAD,), jnp.float32),
        ],
    )
    def deg_kernel(srcb_hbm, dstb_hbm, out_hbm, src_v, dst_v, degs_v, degd_v):
        cid = lax.axis_index("c")
        sid = lax.axis_index("s")
        wid = sid * 2 + cid
        zeros16 = jnp.zeros((16,), jnp.float32)
        ones16 = jnp.ones((16,), jnp.float32)

        @pl.loop(0, N_PAD // 16)
        def _(i):
            degs_v[pl.ds(i * 16, 16)] = zeros16
            degd_v[pl.ds(i * 16, 16)] = zeros16

        pltpu.sync_copy(srcb_hbm.at[wid], src_v)
        pltpu.sync_copy(dstb_hbm.at[wid], dst_v)

        @pl.loop(0, CHUNKS_W)
        def _(j):
            for s2 in range(CHUNK // 16):
                s16 = src_v[j, pl.ds(s2 * 16, 16)]
                plsc.addupdate_scatter(degs_v, [s16], ones16)
                d16 = dst_v[j, pl.ds(s2 * 16, 16)]
                plsc.addupdate_scatter(degd_v, [d16], ones16)

        pltpu.sync_copy(degs_v, out_hbm.at[wid, 0])
        pltpu.sync_copy(degd_v, out_hbm.at[wid, 1])

    return deg_kernel(srcb, dstb)


def _sc_edge_agg(h, srcb, dstb, d):
    @functools.partial(
        pl.kernel,
        out_type=jax.ShapeDtypeStruct((2, N_PAD, d), jnp.float32),
        mesh=_sc_mesh(),
        scratch_types=[
            pltpu.VMEM((PASS_N, CHUNK), jnp.int32),
            pltpu.VMEM((PASS_N, CHUNK), jnp.int32),
            pltpu.VMEM((CHUNK, d), jnp.float32),
            pltpu.VMEM((CHUNK, d), jnp.float32),
            pltpu.VMEM_SHARED((N_PAD, d), jnp.float32),
            pltpu.SemaphoreType.DMA,
            pltpu.SemaphoreType.DMA,
        ],
    )
    def agg_kernel(h_hbm, srcb_hbm, dstb_hbm, out_hbm, src_v, dst_v, msgs_v,
                   msgs_w, agg_sh, sem_a, sem_b):
        cid = lax.axis_index("c")
        sid = lax.axis_index("s")
        zeros16 = jnp.zeros((16,), jnp.float32)
        base = sid * STRIPE

        @pl.loop(0, CHUNK)
        def _(r):
            for s2 in range(d // 16):
                msgs_v[r, pl.ds(s2 * 16, 16)] = zeros16

        for z in range(STRIPE // CHUNK):
            pltpu.sync_copy(msgs_v, agg_sh.at[pl.ds(base + z * CHUNK, CHUNK)])
        plsc.subcore_barrier()

        chunk0 = cid * CORE1_BASE + sid * CH_W
        for p in range(CH_W // PASS_N):
            pltpu.sync_copy(srcb_hbm.at[pl.ds(chunk0 + p * PASS_N, PASS_N)], src_v)
            pltpu.sync_copy(dstb_hbm.at[pl.ds(chunk0 + p * PASS_N, PASS_N)], dst_v)

            pltpu.async_copy(h_hbm.at[src_v.at[0]], msgs_v, sem_a)

            @pl.loop(0, PASS_N // 2)
            def _(k):
                j = 2 * k
                pltpu.make_async_copy(h_hbm.at[src_v.at[j]], msgs_v, sem_a).wait()
                pltpu.async_copy(h_hbm.at[src_v.at[j + 1]], msgs_w, sem_b)
                pltpu.sync_copy(msgs_v, agg_sh.at[dst_v.at[j]], add=True)
                pltpu.make_async_copy(
                    h_hbm.at[src_v.at[j + 1]], msgs_w, sem_b).wait()

                @pl.when(k < PASS_N // 2 - 1)
                def _():
                    pltpu.async_copy(h_hbm.at[src_v.at[j + 2]], msgs_v, sem_a)

                pltpu.sync_copy(msgs_w, agg_sh.at[dst_v.at[j + 1]], add=True)

        plsc.subcore_barrier()
        pltpu.sync_copy(
            agg_sh.at[pl.ds(base, STRIPE)], out_hbm.at[cid, pl.ds(base, STRIPE)]
        )

    return agg_kernel(h, srcb, dstb)


def _norm_from_deg(deg):
    return jnp.where(deg > 0, lax.rsqrt(jnp.maximum(deg, 1e-12)), 0.0)


_DOT = functools.partial(
    lax.dot_general,
    dimension_numbers=(((1,), (0,)), ((), ())),
    precision=lax.Precision.HIGHEST,
    preferred_element_type=jnp.float32,
)

_BLK = 1280


def _tc_prep(xp, degp, W1):
    def body(x_ref, degp_ref, w_ref, out_ref):
        deg = jnp.sum(degp_ref[...], axis=0)
        norm_out = _norm_from_deg(deg[0])
        xs = x_ref[...] * norm_out[:, None]
        out_ref[...] = _DOT(xs, w_ref[...])

    return pl.pallas_call(
        body,
        grid=(N_PAD // _BLK,),
        in_specs=[
            pl.BlockSpec((_BLK, D_IN), lambda i: (i, 0)),
            pl.BlockSpec((NUM_WORKERS, 2, _BLK), lambda i: (0, 0, i)),
            pl.BlockSpec((D_IN, D_H), lambda i: (0, 0)),
        ],
        out_specs=pl.BlockSpec((_BLK, D_H), lambda i: (i, 0)),
        out_shape=jax.ShapeDtypeStruct((N_PAD, D_H), jnp.float32),
    )(xp, degp, W1)


def _tc_mid(aggp, degp, b1, W2):
    def body(aggp_ref, degp_ref, b1_ref, w_ref, out_ref):
        agg = aggp_ref[0] + aggp_ref[1]
        deg = jnp.sum(degp_ref[...], axis=0)
        norm_out = _norm_from_deg(deg[0])
        norm_in = _norm_from_deg(deg[1])
        h = jnp.maximum(agg * norm_in[:, None] + b1_ref[...], 0.0)
        h = h * norm_out[:, None]
        out_ref[...] = _DOT(h, w_ref[...])

    return pl.pallas_call(
        body,
        grid=(N_PAD // _BLK,),
        in_specs=[
            pl.BlockSpec((2, _BLK, D_H), lambda i: (0, i, 0)),
            pl.BlockSpec((NUM_WORKERS, 2, _BLK), lambda i: (0, 0, i)),
            pl.BlockSpec((1, D_H), lambda i: (0, 0)),
            pl.BlockSpec((D_H, D_H), lambda i: (0, 0)),
        ],
        out_specs=pl.BlockSpec((_BLK, D_H), lambda i: (i, 0)),
        out_shape=jax.ShapeDtypeStruct((N_PAD, D_H), jnp.float32),
    )(aggp, degp, b1, W2)


def _tc_final(aggp, degp, b2):
    def body(aggp_ref, degp_ref, b2_ref, out_ref):
        agg = aggp_ref[0, :, :D_OUT] + aggp_ref[1, :, :D_OUT]
        deg = jnp.sum(degp_ref[...], axis=0)
        norm_in = _norm_from_deg(deg[1])
        out_ref[...] = agg * norm_in[:, None] + b2_ref[...]

    return pl.pallas_call(
        body,
        grid=(N_PAD // _BLK,),
        in_specs=[
            pl.BlockSpec((2, _BLK, D_H), lambda i: (0, i, 0)),
            pl.BlockSpec((NUM_WORKERS, 2, _BLK), lambda i: (0, 0, i)),
            pl.BlockSpec((1, D_OUT), lambda i: (0, 0)),
        ],
        out_specs=pl.BlockSpec((_BLK, D_OUT), lambda i: (i, 0)),
        out_shape=jax.ShapeDtypeStruct((N_PAD, D_OUT), jnp.float32),
    )(aggp, degp, b2)


def kernel(features, edge_index, W1, b1, W2, b2):
    xp = jnp.zeros((N_PAD, D_IN), jnp.float32).at[:N_NODES].set(features)
    pad_ids = N_NODES + jnp.arange(E_PAD - N_EDGES, dtype=jnp.int32) % (
        N_PAD - N_NODES)
    ei = jnp.concatenate([edge_index, jnp.stack([pad_ids, pad_ids])], axis=1)
    srcb, dstb = ei[0].reshape(NUM_WORKERS, CHUNKS_W, CHUNK), ei[1].reshape(
        NUM_WORKERS, CHUNKS_W, CHUNK)
    srcf, dstf = ei[0].reshape(-1, CHUNK), ei[1].reshape(-1, CHUNK)

    W2p = jnp.concatenate([W2, jnp.zeros((D_H, D_H - D_OUT), jnp.float32)], axis=1)

    degp = _sc_degrees(srcb, dstb)
    h1p = _tc_prep(xp, degp, W1)
    agg1 = _sc_edge_agg(h1p, srcf, dstf, D_H)
    h2p = _tc_mid(agg1, degp, b1.reshape(1, D_H), W2p)
    agg2 = _sc_edge_agg(h2p, srcf, dstf, D_H)
    outp = _tc_final(agg2, degp, b2.reshape(1, D_OUT))
    return outp[:N_NODES]
